# baseline (speedup 1.0000x reference)
import jax
import jax.numpy as jnp
from jax.experimental import pallas as pl
from jax.experimental.pallas import tpu as pltpu

_N = 10000
_C = 128
_ROWS = 50


def _body(z_hbm, x_hbm, o_hbm, sem):
    del z_hbm
    c1 = pltpu.make_async_copy(
        x_hbm.at[:, pl.ds(0, 1), :], o_hbm.at[:, pl.ds(0, 1), :], sem.at[0]
    )
    c2 = pltpu.make_async_copy(
        x_hbm.at[:, pl.ds(1, 1), :], o_hbm.at[:, pl.ds(25, 1), :], sem.at[1]
    )
    c1.start()
    c2.start()
    c1.wait()
    c2.wait()


def kernel(atom_embeddings):
    z = jnp.zeros((_N, _ROWS, _C), atom_embeddings.dtype)
    x3 = atom_embeddings.reshape(_N, 2, _C)
    return pl.pallas_call(
        _body,
        in_specs=[
            pl.BlockSpec(memory_space=pltpu.MemorySpace.HBM),
            pl.BlockSpec(memory_space=pltpu.MemorySpace.HBM),
        ],
        out_specs=pl.BlockSpec(memory_space=pltpu.MemorySpace.HBM),
        out_shape=jax.ShapeDtypeStruct((_N, _ROWS, _C), atom_embeddings.dtype),
        input_output_aliases={0: 0},
        scratch_shapes=[pltpu.SemaphoreType.DMA((2,))],
    )(z, x3)


# speedup vs baseline: 2.0957x; 2.0957x over previous
import jax
import jax.numpy as jnp
from jax.experimental import pallas as pl
from jax.experimental.pallas import tpu as pltpu

_N = 10000
_C = 128
_ROWS = 50
_B = 250
_NB = _N // _B


def _body(z_hbm, x_hbm, o_hbm, xv, isem, dsem):
    del z_hbm
    ic = pltpu.make_async_copy(x_hbm, xv, isem)
    ic.start()
    ic.wait()
    cs = []
    for b in range(_NB):
        a0 = b * _B
        for j, r0 in enumerate((0, 25)):
            c = pltpu.make_async_copy(
                xv.at[pl.ds(a0, _B), pl.ds(j, 1), :],
                o_hbm.at[pl.ds(a0, _B), pl.ds(r0, 1), :],
                dsem.at[b, j],
            )
            c.start()
            cs.append(c)
    for c in cs:
        c.wait()


def kernel(atom_embeddings):
    z = jnp.zeros((_N, _ROWS, _C), atom_embeddings.dtype)
    x3 = atom_embeddings.reshape(_N, 2, _C)
    return pl.pallas_call(
        _body,
        in_specs=[
            pl.BlockSpec(memory_space=pltpu.MemorySpace.HBM),
            pl.BlockSpec(memory_space=pltpu.MemorySpace.HBM),
        ],
        out_specs=pl.BlockSpec(memory_space=pltpu.MemorySpace.HBM),
        out_shape=jax.ShapeDtypeStruct((_N, _ROWS, _C), atom_embeddings.dtype),
        input_output_aliases={0: 0},
        scratch_shapes=[
            pltpu.VMEM((_N, 2, _C), jnp.float32),
            pltpu.SemaphoreType.DMA,
            pltpu.SemaphoreType.DMA((_NB, 2)),
        ],
    )(z, x3)
